# baseline (speedup 1.0000x reference)
import functools

import jax
import jax.numpy as jnp
from jax import lax
from jax.experimental import pallas as pl
from jax.experimental.pallas import tpu as pltpu
from jax.experimental.pallas import tpu_sc as plsc

N = 100000
D = 512
S = 512
NL = 16
NV = D // NL
CH = 40
ROWS_W = 3120
NPAIR = ROWS_W // (2 * CH)
NPAIR_LAST = (N - 31 * ROWS_W) // (2 * CH)
NW = 32


def _lane_sum(v):
    lanes = lax.iota(jnp.int32, NL)
    for k in (1, 2, 4, 8):
        v = v + v.at[lanes ^ k].get(mode="promise_in_bounds")
    return v


def _mesh():
    return plsc.VectorSubcoreMesh(core_axis_name="c", subcore_axis_name="s")


def _worker():
    cid = lax.axis_index("c")
    sid = lax.axis_index("s")
    wid = cid * 16 + sid
    base = wid * ROWS_W
    npairs = jnp.where(wid == NW - 1, NPAIR_LAST, NPAIR)
    return wid, base, npairs


def _start_chunk(x_hbm, b_hbm, rowbase, xbuf, idxbuf, sem):
    pltpu.async_copy(x_hbm.at[pl.ds(rowbase, CH)], xbuf, sem)
    pltpu.async_copy(b_hbm.at[pl.ds(rowbase, CH)], idxbuf.at[pl.ds(0, CH)],
                     sem)


def _wait_chunk(x_hbm, b_hbm, rowbase, xbuf, idxbuf, sem):
    pltpu.make_async_copy(x_hbm.at[pl.ds(rowbase, CH)], xbuf, sem).wait()
    pltpu.make_async_copy(b_hbm.at[pl.ds(rowbase, CH)],
                          idxbuf.at[pl.ds(0, CH)], sem).wait()


def _sc_pass1_call(x, batch, zero16):
    @functools.partial(
        pl.kernel,
        out_type=(
            jax.ShapeDtypeStruct((NW, S, D), jnp.float32),
            jax.ShapeDtypeStruct((NW, S * NL), jnp.float32),
        ),
        mesh=_mesh(),
        scratch_types=[
            pltpu.VMEM((CH, D), jnp.float32),
            pltpu.VMEM((CH, D), jnp.float32),
            pltpu.VMEM((CH + NL,), jnp.int32),
            pltpu.VMEM((CH + NL,), jnp.int32),
            pltpu.VMEM((D,), jnp.float32),
            pltpu.VMEM((S * NL,), jnp.float32),
            pltpu.SemaphoreType.DMA,
            pltpu.SemaphoreType.DMA,
        ],
    )
    def k(x_hbm, b_hbm, z16_hbm, psum_hbm, pcnt_hbm,
          xbuf0, xbuf1, idx0, idx1, sacc, cntv, sem0, sem1):
        wid, base, npairs = _worker()
        pltpu.sync_copy(z16_hbm, cntv)
        pltpu.sync_copy(z16_hbm.at[pl.ds(0, D)], sacc)

        def compute(xbuf, idxbuf, carry):
            def row(r, rcarry):
                prev_id, cnt = rcarry
                rid = idxbuf[pl.ds(r, NL)][0]
                change = rid != prev_id

                @pl.when(jnp.logical_and(change, prev_id >= 0))
                def _():
                    cntv[pl.ds(prev_id * NL, NL)] = cnt
                    pltpu.sync_copy(sacc, psum_hbm.at[wid, prev_id])
                    for c in range(NV):
                        sacc[pl.ds(c * NL, NL)] = jnp.zeros((NL,),
                                                            jnp.float32)

                for c in range(NV):
                    plsc.addupdate(sacc.at[pl.ds(c * NL, NL)],
                                   xbuf[r, pl.ds(c * NL, NL)])
                new_cnt = jnp.where(change, 1.0, cnt + 1.0)
                return (rid, new_cnt)

            return lax.fori_loop(0, CH, row, carry)

        def pair(g, carry):
            rb0 = base + (2 * g) * CH
            rb1 = rb0 + CH
            _start_chunk(x_hbm, b_hbm, rb1, xbuf1, idx1, sem1)
            _wait_chunk(x_hbm, b_hbm, rb0, xbuf0, idx0, sem0)
            carry = compute(xbuf0, idx0, carry)

            @pl.when(g + 1 < npairs)
            def _():
                _start_chunk(x_hbm, b_hbm, rb1 + CH, xbuf0, idx0, sem0)

            _wait_chunk(x_hbm, b_hbm, rb1, xbuf1, idx1, sem1)
            return compute(xbuf1, idx1, carry)

        _start_chunk(x_hbm, b_hbm, base, xbuf0, idx0, sem0)
        init = (jnp.int32(-1), jnp.zeros((NL,), jnp.float32))
        prev_id, cnt = lax.fori_loop(0, npairs, pair, init)

        @pl.when(prev_id >= 0)
        def _():
            cntv[pl.ds(prev_id * NL, NL)] = cnt
            pltpu.sync_copy(sacc, psum_hbm.at[wid, prev_id])

        pltpu.sync_copy(cntv, pcnt_hbm.at[wid])

    return k(x, batch, zero16)


def _tc_dense_call(psum, pcnt, W):
    def body(psum_ref, pcnt_ref, w_ref, t_ref, acc, cntacc):
        t = pl.program_id(0)

        @pl.when(t == 0)
        def _():
            acc[...] = jnp.zeros_like(acc)
            cntacc[...] = jnp.zeros_like(cntacc)

        c = pcnt_ref[0, :, 0]
        valid = c > 0.0
        acc[...] += jnp.where(valid[:, None], psum_ref[0], 0.0)
        cntacc[...] += jnp.where(valid, c, 0.0)

        @pl.when(t == NW - 1)
        def _():
            cnt = jnp.maximum(cntacc[...], 1.0)
            mean = acc[...] / cnt[:, None]
            t_ref[...] = jnp.tanh(
                jnp.dot(mean, w_ref[...], preferred_element_type=jnp.float32))

    return pl.pallas_call(
        body,
        grid=(NW,),
        in_specs=[
            pl.BlockSpec((1, S, D), lambda t: (t, 0, 0)),
            pl.BlockSpec((1, S, NL), lambda t: (t, 0, 0)),
            pl.BlockSpec((S, D), lambda t: (0, 0)),
        ],
        out_specs=pl.BlockSpec((S, D), lambda t: (0, 0)),
        out_shape=jax.ShapeDtypeStruct((S, D), jnp.float32),
        scratch_shapes=[
            pltpu.VMEM((S, D), jnp.float32),
            pltpu.VMEM((S,), jnp.float32),
        ],
    )(psum, pcnt, W)


def _sc_pass2_call(x, batch, t):
    @functools.partial(
        pl.kernel,
        out_type=jax.ShapeDtypeStruct((NW, S, D), jnp.float32),
        mesh=_mesh(),
        scratch_types=[
            pltpu.VMEM((CH, D), jnp.float32),
            pltpu.VMEM((CH, D), jnp.float32),
            pltpu.VMEM((CH + NL,), jnp.int32),
            pltpu.VMEM((CH + NL,), jnp.int32),
            pltpu.VMEM((D,), jnp.float32),
            pltpu.VMEM((D,), jnp.float32),
            pltpu.VMEM((CH * NL,), jnp.float32),
            pltpu.SemaphoreType.DMA,
            pltpu.SemaphoreType.DMA,
        ],
    )
    def k(x_hbm, b_hbm, t_hbm, out_hbm,
          xbuf0, xbuf1, idx0, idx1, wacc, trowv, cfbuf, sem0, sem1):
        wid, base, npairs = _worker()
        for c in range(NV):
            wacc[pl.ds(c * NL, NL)] = jnp.zeros((NL,), jnp.float32)

        def dot_gate(dacc):
            zv = _lane_sum(dacc * (-10.0))
            return 1.0 / (1.0 + jnp.exp(zv))

        def phase_a(xbuf, idxbuf, prev_id):
            for gs, gn in ((0, NL), (NL, NL), (2 * NL, CH - 2 * NL)):
                idv = idxbuf[pl.ds(gs, NL)]
                first = idv[0]
                last = idv[gn - 1]
                pid = prev_id

                def fast(first=first, last=last, pid=pid, gs=gs, gn=gn,
                         xbuf=xbuf):
                    @pl.when(first != pid)
                    def _():
                        pltpu.sync_copy(t_hbm.at[first], trowv)

                    tr = [trowv[pl.ds(c * NL, NL)] for c in range(NV)]

                    def rowfn(r, z):
                        dacc = xbuf[r, pl.ds(0, NL)] * tr[0]
                        for c in range(1, NV):
                            dacc = dacc + xbuf[r, pl.ds(c * NL, NL)] * tr[c]
                        cfbuf[pl.ds(r * NL, NL)] = dot_gate(dacc)
                        return z

                    lax.fori_loop(gs, gs + gn, rowfn, 0)
                    return last

                def slow(pid=pid, gs=gs, gn=gn, xbuf=xbuf, idxbuf=idxbuf):
                    def rowfn(r, p):
                        rid = idxbuf[pl.ds(r, NL)][0]

                        @pl.when(rid != p)
                        def _():
                            pltpu.sync_copy(t_hbm.at[rid], trowv)

                        dacc = (xbuf[r, pl.ds(0, NL)]
                                * trowv[pl.ds(0, NL)])
                        for c in range(1, NV):
                            dacc = dacc + (xbuf[r, pl.ds(c * NL, NL)]
                                           * trowv[pl.ds(c * NL, NL)])
                        cfbuf[pl.ds(r * NL, NL)] = dot_gate(dacc)
                        return rid

                    return lax.fori_loop(gs, gs + gn, rowfn, pid)

                prev_id = lax.cond(first == last, fast, slow)
            return prev_id

        def phase_b(xbuf, idxbuf, prev_id):
            def row(r, pid):
                rid = idxbuf[pl.ds(r, NL)][0]

                @pl.when(jnp.logical_and(rid != pid, pid >= 0))
                def _():
                    pltpu.sync_copy(wacc, out_hbm.at[wid, pid])
                    for c in range(NV):
                        wacc[pl.ds(c * NL, NL)] = jnp.zeros((NL,),
                                                            jnp.float32)

                cf = cfbuf[pl.ds(r * NL, NL)]
                for c in range(NV):
                    plsc.addupdate(wacc.at[pl.ds(c * NL, NL)],
                                   xbuf[r, pl.ds(c * NL, NL)] * cf)
                return rid

            return lax.fori_loop(0, CH, row, prev_id)

        def compute(xbuf, idxbuf, carry):
            ca, cb = carry
            return (phase_a(xbuf, idxbuf, ca), phase_b(xbuf, idxbuf, cb))

        def pair(g, carry):
            rb0 = base + (2 * g) * CH
            rb1 = rb0 + CH
            _start_chunk(x_hbm, b_hbm, rb1, xbuf1, idx1, sem1)
            _wait_chunk(x_hbm, b_hbm, rb0, xbuf0, idx0, sem0)
            carry = compute(xbuf0, idx0, carry)

            @pl.when(g + 1 < npairs)
            def _():
                _start_chunk(x_hbm, b_hbm, rb1 + CH, xbuf0, idx0, sem0)

            _wait_chunk(x_hbm, b_hbm, rb1, xbuf1, idx1, sem1)
            return compute(xbuf1, idx1, carry)

        _start_chunk(x_hbm, b_hbm, base, xbuf0, idx0, sem0)
        init = (jnp.int32(-1), jnp.int32(-1))
        carry_a, prev_id = lax.fori_loop(0, npairs, pair, init)

        @pl.when(prev_id >= 0)
        def _():
            pltpu.sync_copy(wacc, out_hbm.at[wid, prev_id])

    return k(x, batch, t)


def _tc_combine_call(parts, pcnt):
    def body(parts_ref, pcnt_ref, out_ref, acc):
        t = pl.program_id(0)

        @pl.when(t == 0)
        def _():
            acc[...] = jnp.zeros_like(acc)

        valid = pcnt_ref[0, :, 0] > 0.0
        acc[...] += jnp.where(valid[:, None], parts_ref[0], 0.0)

        @pl.when(t == NW - 1)
        def _():
            out_ref[...] = acc[...]

    return pl.pallas_call(
        body,
        grid=(NW,),
        in_specs=[
            pl.BlockSpec((1, S, D), lambda t: (t, 0, 0)),
            pl.BlockSpec((1, S, NL), lambda t: (t, 0, 0)),
        ],
        out_specs=pl.BlockSpec((S, D), lambda t: (0, 0)),
        out_shape=jax.ShapeDtypeStruct((S, D), jnp.float32),
        scratch_shapes=[pltpu.VMEM((S, D), jnp.float32)],
    )(parts, pcnt)


def kernel(x, batch, size, W):
    del size
    batch = batch.astype(jnp.int32)
    zero16 = jnp.zeros((S * NL,), jnp.float32)
    psum, pcnt = _sc_pass1_call(x, batch, zero16)
    pcnt = pcnt.reshape(NW, S, NL)
    t = _tc_dense_call(psum, pcnt, W)
    parts = _sc_pass2_call(x, batch, t)
    return _tc_combine_call(parts, pcnt)


# speedup vs baseline: 1.9079x; 1.9079x over previous
import functools

import jax
import jax.numpy as jnp
from jax import lax
from jax.experimental import pallas as pl
from jax.experimental.pallas import tpu as pltpu
from jax.experimental.pallas import tpu_sc as plsc

N = 100000
D = 512
S = 512
NL = 16
NV = D // NL
CH = 40
ROWS_W = 3120
NPAIR = ROWS_W // (2 * CH)
NPAIR_LAST = (N - 31 * ROWS_W) // (2 * CH)
NW = 32


def _lane_sum(v):
    lanes = lax.iota(jnp.int32, NL)
    for k in (1, 2, 4, 8):
        v = v + v.at[lanes ^ k].get(mode="promise_in_bounds")
    return v


def _mesh():
    return plsc.VectorSubcoreMesh(core_axis_name="c", subcore_axis_name="s")


def _worker():
    cid = lax.axis_index("c")
    sid = lax.axis_index("s")
    wid = cid * 16 + sid
    base = wid * ROWS_W
    npairs = jnp.where(wid == NW - 1, NPAIR_LAST, NPAIR)
    return wid, base, npairs


def _start_chunk(x_hbm, b_hbm, rowbase, xbuf, idxbuf, sem):
    pltpu.async_copy(x_hbm.at[pl.ds(rowbase, CH)], xbuf, sem)
    pltpu.async_copy(b_hbm.at[pl.ds(rowbase, CH)], idxbuf.at[pl.ds(0, CH)],
                     sem)


def _wait_chunk(x_hbm, b_hbm, rowbase, xbuf, idxbuf, sem):
    pltpu.make_async_copy(x_hbm.at[pl.ds(rowbase, CH)], xbuf, sem).wait()
    pltpu.make_async_copy(b_hbm.at[pl.ds(rowbase, CH)],
                          idxbuf.at[pl.ds(0, CH)], sem).wait()


def _sc_pass1_call(x, batch, zero16):
    @functools.partial(
        pl.kernel,
        out_type=(
            jax.ShapeDtypeStruct((NW, S, D), jnp.float32),
            jax.ShapeDtypeStruct((NW, S * NL), jnp.float32),
        ),
        mesh=_mesh(),
        scratch_types=[
            pltpu.VMEM((CH, D), jnp.float32),
            pltpu.VMEM((CH, D), jnp.float32),
            pltpu.VMEM((CH + NL,), jnp.int32),
            pltpu.VMEM((CH + NL,), jnp.int32),
            pltpu.VMEM((D,), jnp.float32),
            pltpu.VMEM((S * NL,), jnp.float32),
            pltpu.SemaphoreType.DMA,
            pltpu.SemaphoreType.DMA,
        ],
    )
    def k(x_hbm, b_hbm, z16_hbm, psum_hbm, pcnt_hbm,
          xbuf0, xbuf1, idx0, idx1, stg, cntv, sem0, sem1):
        wid, base, npairs = _worker()
        pltpu.sync_copy(z16_hbm, cntv)

        def flush(prev_id, accs, cnt):
            for c in range(NV):
                stg[pl.ds(c * NL, NL)] = accs[c]
            cntv[pl.ds(prev_id * NL, NL)] = cnt
            pltpu.sync_copy(stg, psum_hbm.at[wid, prev_id])

        def compute(xbuf, idxbuf, carry):
            def row(r, rcarry):
                prev_id, cnt, *accs = rcarry
                rid = idxbuf[pl.ds(r, NL)][0]
                change = rid != prev_id

                @pl.when(jnp.logical_and(change, prev_id >= 0))
                def _():
                    flush(prev_id, accs, cnt)

                xs = [xbuf[r, pl.ds(c * NL, NL)] for c in range(NV)]
                new_accs = [
                    jnp.where(change, xs[c], accs[c] + xs[c])
                    for c in range(NV)
                ]
                new_cnt = jnp.where(change, 1.0, cnt + 1.0)
                return (rid, new_cnt, *new_accs)

            return lax.fori_loop(0, CH, row, carry)

        def pair(g, carry):
            rb0 = base + (2 * g) * CH
            rb1 = rb0 + CH
            _start_chunk(x_hbm, b_hbm, rb1, xbuf1, idx1, sem1)
            _wait_chunk(x_hbm, b_hbm, rb0, xbuf0, idx0, sem0)
            carry = compute(xbuf0, idx0, carry)

            @pl.when(g + 1 < npairs)
            def _():
                _start_chunk(x_hbm, b_hbm, rb1 + CH, xbuf0, idx0, sem0)

            _wait_chunk(x_hbm, b_hbm, rb1, xbuf1, idx1, sem1)
            return compute(xbuf1, idx1, carry)

        _start_chunk(x_hbm, b_hbm, base, xbuf0, idx0, sem0)
        init = (jnp.int32(-1), jnp.zeros((NL,), jnp.float32)) + tuple(
            jnp.zeros((NL,), jnp.float32) for _ in range(NV))
        prev_id, cnt, *accs = lax.fori_loop(0, npairs, pair, init)

        @pl.when(prev_id >= 0)
        def _():
            flush(prev_id, accs, cnt)

        pltpu.sync_copy(cntv, pcnt_hbm.at[wid])

    return k(x, batch, zero16)


def _tc_dense_call(psum, pcnt, W):
    def body(psum_ref, pcnt_ref, w_ref, t_ref, acc, cntacc):
        t = pl.program_id(0)

        @pl.when(t == 0)
        def _():
            acc[...] = jnp.zeros_like(acc)
            cntacc[...] = jnp.zeros_like(cntacc)

        c = pcnt_ref[0, :, 0]
        valid = c > 0.0
        acc[...] += jnp.where(valid[:, None], psum_ref[0], 0.0)
        cntacc[...] += jnp.where(valid, c, 0.0)

        @pl.when(t == NW - 1)
        def _():
            cnt = jnp.maximum(cntacc[...], 1.0)
            mean = acc[...] / cnt[:, None]
            t_ref[...] = jnp.tanh(
                jnp.dot(mean, w_ref[...], preferred_element_type=jnp.float32))

    return pl.pallas_call(
        body,
        grid=(NW,),
        in_specs=[
            pl.BlockSpec((1, S, D), lambda t: (t, 0, 0)),
            pl.BlockSpec((1, S, NL), lambda t: (t, 0, 0)),
            pl.BlockSpec((S, D), lambda t: (0, 0)),
        ],
        out_specs=pl.BlockSpec((S, D), lambda t: (0, 0)),
        out_shape=jax.ShapeDtypeStruct((S, D), jnp.float32),
        scratch_shapes=[
            pltpu.VMEM((S, D), jnp.float32),
            pltpu.VMEM((S,), jnp.float32),
        ],
    )(psum, pcnt, W)


def _sc_pass2_call(x, batch, t):
    @functools.partial(
        pl.kernel,
        out_type=jax.ShapeDtypeStruct((NW, S, D), jnp.float32),
        mesh=_mesh(),
        scratch_types=[
            pltpu.VMEM((CH, D), jnp.float32),
            pltpu.VMEM((CH, D), jnp.float32),
            pltpu.VMEM((CH + NL,), jnp.int32),
            pltpu.VMEM((CH + NL,), jnp.int32),
            pltpu.VMEM((D,), jnp.float32),
            pltpu.VMEM((D,), jnp.float32),
            pltpu.VMEM((CH * NL,), jnp.float32),
            pltpu.SemaphoreType.DMA,
            pltpu.SemaphoreType.DMA,
        ],
    )
    def k(x_hbm, b_hbm, t_hbm, out_hbm,
          xbuf0, xbuf1, idx0, idx1, stg, trowv, cfbuf, sem0, sem1):
        wid, base, npairs = _worker()

        def dot_gate(dacc):
            zv = _lane_sum(dacc * (-10.0))
            return 1.0 / (1.0 + jnp.exp(zv))

        def flush(prev_id, accs):
            for c in range(NV):
                stg[pl.ds(c * NL, NL)] = accs[c]
            pltpu.sync_copy(stg, out_hbm.at[wid, prev_id])

        def phase_a(xbuf, idxbuf, prev_id):
            for gs, gn in ((0, NL), (NL, NL), (2 * NL, CH - 2 * NL)):
                idv = idxbuf[pl.ds(gs, NL)]
                first = idv[0]
                last = idv[gn - 1]
                pid = prev_id

                def fast(first=first, last=last, pid=pid, gs=gs, gn=gn,
                         xbuf=xbuf):
                    @pl.when(first != pid)
                    def _():
                        pltpu.sync_copy(t_hbm.at[first], trowv)

                    tr = [trowv[pl.ds(c * NL, NL)] for c in range(NV)]

                    def rowfn(r, z):
                        dacc = xbuf[r, pl.ds(0, NL)] * tr[0]
                        for c in range(1, NV):
                            dacc = dacc + xbuf[r, pl.ds(c * NL, NL)] * tr[c]
                        cfbuf[pl.ds(r * NL, NL)] = dot_gate(dacc)
                        return z

                    lax.fori_loop(gs, gs + gn, rowfn, 0)
                    return last

                def slow(pid=pid, gs=gs, gn=gn, xbuf=xbuf, idxbuf=idxbuf):
                    def rowfn(r, p):
                        rid = idxbuf[pl.ds(r, NL)][0]

                        @pl.when(rid != p)
                        def _():
                            pltpu.sync_copy(t_hbm.at[rid], trowv)

                        dacc = (xbuf[r, pl.ds(0, NL)]
                                * trowv[pl.ds(0, NL)])
                        for c in range(1, NV):
                            dacc = dacc + (xbuf[r, pl.ds(c * NL, NL)]
                                           * trowv[pl.ds(c * NL, NL)])
                        cfbuf[pl.ds(r * NL, NL)] = dot_gate(dacc)
                        return rid

                    return lax.fori_loop(gs, gs + gn, rowfn, pid)

                prev_id = lax.cond(first == last, fast, slow)
            return prev_id

        def phase_b(xbuf, idxbuf, carry):
            def row(r, rcarry):
                prev_id, *accs = rcarry
                rid = idxbuf[pl.ds(r, NL)][0]
                change = rid != prev_id

                @pl.when(jnp.logical_and(change, prev_id >= 0))
                def _():
                    flush(prev_id, accs)

                cf = cfbuf[pl.ds(r * NL, NL)]
                new_accs = []
                for c in range(NV):
                    wx = xbuf[r, pl.ds(c * NL, NL)] * cf
                    new_accs.append(jnp.where(change, wx, accs[c] + wx))
                return (rid, *new_accs)

            return lax.fori_loop(0, CH, row, carry)

        def compute(xbuf, idxbuf, carry):
            ca, cb = carry
            return (phase_a(xbuf, idxbuf, ca), phase_b(xbuf, idxbuf, cb))

        def pair(g, carry):
            rb0 = base + (2 * g) * CH
            rb1 = rb0 + CH
            _start_chunk(x_hbm, b_hbm, rb1, xbuf1, idx1, sem1)
            _wait_chunk(x_hbm, b_hbm, rb0, xbuf0, idx0, sem0)
            carry = compute(xbuf0, idx0, carry)

            @pl.when(g + 1 < npairs)
            def _():
                _start_chunk(x_hbm, b_hbm, rb1 + CH, xbuf0, idx0, sem0)

            _wait_chunk(x_hbm, b_hbm, rb1, xbuf1, idx1, sem1)
            return compute(xbuf1, idx1, carry)

        _start_chunk(x_hbm, b_hbm, base, xbuf0, idx0, sem0)
        zv16 = jnp.zeros((NL,), jnp.float32)
        init_b = (jnp.int32(-1),) + tuple(zv16 for _ in range(NV))
        carry_a, carry_b = lax.fori_loop(
            0, npairs, pair, (jnp.int32(-1), init_b))
        prev_id, *accs = carry_b

        @pl.when(prev_id >= 0)
        def _():
            flush(prev_id, accs)

    return k(x, batch, t)


def _tc_combine_call(parts, pcnt):
    def body(parts_ref, pcnt_ref, out_ref, acc):
        t = pl.program_id(0)

        @pl.when(t == 0)
        def _():
            acc[...] = jnp.zeros_like(acc)

        valid = pcnt_ref[0, :, 0] > 0.0
        acc[...] += jnp.where(valid[:, None], parts_ref[0], 0.0)

        @pl.when(t == NW - 1)
        def _():
            out_ref[...] = acc[...]

    return pl.pallas_call(
        body,
        grid=(NW,),
        in_specs=[
            pl.BlockSpec((1, S, D), lambda t: (t, 0, 0)),
            pl.BlockSpec((1, S, NL), lambda t: (t, 0, 0)),
        ],
        out_specs=pl.BlockSpec((S, D), lambda t: (0, 0)),
        out_shape=jax.ShapeDtypeStruct((S, D), jnp.float32),
        scratch_shapes=[pltpu.VMEM((S, D), jnp.float32)],
    )(parts, pcnt)


def kernel(x, batch, size, W):
    del size
    batch = batch.astype(jnp.int32)
    zero16 = jnp.zeros((S * NL,), jnp.float32)
    psum, pcnt = _sc_pass1_call(x, batch, zero16)
    pcnt = pcnt.reshape(NW, S, NL)
    t = _tc_dense_call(psum, pcnt, W)
    parts = _sc_pass2_call(x, batch, t)
    return _tc_combine_call(parts, pcnt)


# speedup vs baseline: 2.1773x; 1.1412x over previous
import functools

import jax
import jax.numpy as jnp
from jax import lax
from jax.experimental import pallas as pl
from jax.experimental.pallas import tpu as pltpu
from jax.experimental.pallas import tpu_sc as plsc

N = 100000
D = 512
S = 512
NL = 16
NV = D // NL
CH = 40
ROWS_W = 3120
NPAIR = ROWS_W // (2 * CH)
NPAIR_LAST = (N - 31 * ROWS_W) // (2 * CH)
NW = 32


def _lane_sum(v):
    lanes = lax.iota(jnp.int32, NL)
    for k in (1, 2, 4, 8):
        v = v + v.at[lanes ^ k].get(mode="promise_in_bounds")
    return v


def _mesh():
    return plsc.VectorSubcoreMesh(core_axis_name="c", subcore_axis_name="s")


def _worker():
    cid = lax.axis_index("c")
    sid = lax.axis_index("s")
    wid = cid * 16 + sid
    base = wid * ROWS_W
    npairs = jnp.where(wid == NW - 1, NPAIR_LAST, NPAIR)
    return wid, base, npairs


def _start_chunk(x_hbm, b_hbm, rowbase, xbuf, idxbuf, sem):
    pltpu.async_copy(x_hbm.at[pl.ds(rowbase, CH)], xbuf, sem)
    pltpu.async_copy(b_hbm.at[pl.ds(rowbase, CH)], idxbuf.at[pl.ds(0, CH)],
                     sem)


def _wait_chunk(x_hbm, b_hbm, rowbase, xbuf, idxbuf, sem):
    pltpu.make_async_copy(x_hbm.at[pl.ds(rowbase, CH)], xbuf, sem).wait()
    pltpu.make_async_copy(b_hbm.at[pl.ds(rowbase, CH)],
                          idxbuf.at[pl.ds(0, CH)], sem).wait()


def _sc_pass1_call(x, batch, zero16):
    @functools.partial(
        pl.kernel,
        out_type=(
            jax.ShapeDtypeStruct((NW, S, D), jnp.float32),
            jax.ShapeDtypeStruct((NW, S * NL), jnp.float32),
        ),
        mesh=_mesh(),
        scratch_types=[
            pltpu.VMEM((CH, D), jnp.float32),
            pltpu.VMEM((CH, D), jnp.float32),
            pltpu.VMEM((CH + NL,), jnp.int32),
            pltpu.VMEM((CH + NL,), jnp.int32),
            pltpu.VMEM((D,), jnp.float32),
            pltpu.VMEM((S * NL,), jnp.float32),
            pltpu.SemaphoreType.DMA,
            pltpu.SemaphoreType.DMA,
        ],
    )
    def k(x_hbm, b_hbm, z16_hbm, psum_hbm, pcnt_hbm,
          xbuf0, xbuf1, idx0, idx1, stg, cntv, sem0, sem1):
        wid, base, npairs = _worker()
        pltpu.sync_copy(z16_hbm, cntv)

        def flush(prev_id, accs, cnt):
            for c in range(NV):
                stg[pl.ds(c * NL, NL)] = accs[c]
            cntv[pl.ds(prev_id * NL, NL)] = cnt
            pltpu.sync_copy(stg, psum_hbm.at[wid, prev_id])

        def compute(xbuf, idxbuf, carry):
            def row(r, rcarry):
                prev_id, cnt, *accs = rcarry
                rid = idxbuf[pl.ds(r, NL)][0]
                change = rid != prev_id

                @pl.when(jnp.logical_and(change, prev_id >= 0))
                def _():
                    flush(prev_id, accs, cnt)

                xs = [xbuf[r, pl.ds(c * NL, NL)] for c in range(NV)]
                new_accs = [
                    jnp.where(change, xs[c], accs[c] + xs[c])
                    for c in range(NV)
                ]
                new_cnt = jnp.where(change, 1.0, cnt + 1.0)
                return (rid, new_cnt, *new_accs)

            return lax.fori_loop(0, CH, row, carry, unroll=2)

        def pair(g, carry):
            rb0 = base + (2 * g) * CH
            rb1 = rb0 + CH
            _start_chunk(x_hbm, b_hbm, rb1, xbuf1, idx1, sem1)
            _wait_chunk(x_hbm, b_hbm, rb0, xbuf0, idx0, sem0)
            carry = compute(xbuf0, idx0, carry)

            @pl.when(g + 1 < npairs)
            def _():
                _start_chunk(x_hbm, b_hbm, rb1 + CH, xbuf0, idx0, sem0)

            _wait_chunk(x_hbm, b_hbm, rb1, xbuf1, idx1, sem1)
            return compute(xbuf1, idx1, carry)

        _start_chunk(x_hbm, b_hbm, base, xbuf0, idx0, sem0)
        init = (jnp.int32(-1), jnp.zeros((NL,), jnp.float32)) + tuple(
            jnp.zeros((NL,), jnp.float32) for _ in range(NV))
        prev_id, cnt, *accs = lax.fori_loop(0, npairs, pair, init)

        @pl.when(prev_id >= 0)
        def _():
            flush(prev_id, accs, cnt)

        pltpu.sync_copy(cntv, pcnt_hbm.at[wid])

    return k(x, batch, zero16)


def _tc_dense_call(psum, pcnt, W):
    def body(psum_ref, pcnt_ref, w_ref, t_ref, acc, cntacc):
        t = pl.program_id(0)

        @pl.when(t == 0)
        def _():
            acc[...] = jnp.zeros_like(acc)
            cntacc[...] = jnp.zeros_like(cntacc)

        c = pcnt_ref[0, :, 0]
        valid = c > 0.0
        acc[...] += jnp.where(valid[:, None], psum_ref[0], 0.0)
        cntacc[...] += jnp.where(valid, c, 0.0)

        @pl.when(t == NW - 1)
        def _():
            cnt = jnp.maximum(cntacc[...], 1.0)
            mean = acc[...] / cnt[:, None]
            t_ref[...] = jnp.tanh(
                jnp.dot(mean, w_ref[...], preferred_element_type=jnp.float32))

    return pl.pallas_call(
        body,
        grid=(NW,),
        in_specs=[
            pl.BlockSpec((1, S, D), lambda t: (t, 0, 0)),
            pl.BlockSpec((1, S, NL), lambda t: (t, 0, 0)),
            pl.BlockSpec((S, D), lambda t: (0, 0)),
        ],
        out_specs=pl.BlockSpec((S, D), lambda t: (0, 0)),
        out_shape=jax.ShapeDtypeStruct((S, D), jnp.float32),
        scratch_shapes=[
            pltpu.VMEM((S, D), jnp.float32),
            pltpu.VMEM((S,), jnp.float32),
        ],
    )(psum, pcnt, W)


def _sc_pass2_call(x, batch, t):
    @functools.partial(
        pl.kernel,
        out_type=jax.ShapeDtypeStruct((NW, S, D), jnp.float32),
        mesh=_mesh(),
        scratch_types=[
            pltpu.VMEM((CH, D), jnp.float32),
            pltpu.VMEM((CH, D), jnp.float32),
            pltpu.VMEM((CH + NL,), jnp.int32),
            pltpu.VMEM((CH + NL,), jnp.int32),
            pltpu.VMEM((D,), jnp.float32),
            pltpu.VMEM((D,), jnp.float32),
            pltpu.VMEM((CH * NL,), jnp.float32),
            pltpu.SemaphoreType.DMA,
            pltpu.SemaphoreType.DMA,
        ],
    )
    def k(x_hbm, b_hbm, t_hbm, out_hbm,
          xbuf0, xbuf1, idx0, idx1, stg, trowv, cfbuf, sem0, sem1):
        wid, base, npairs = _worker()

        def dot_gate(dacc):
            zv = _lane_sum(dacc * (-10.0))
            return 1.0 / (1.0 + jnp.exp(zv))

        def flush(prev_id, accs):
            for c in range(NV):
                stg[pl.ds(c * NL, NL)] = accs[c]
            pltpu.sync_copy(stg, out_hbm.at[wid, prev_id])

        def phase_a(xbuf, idxbuf, prev_id):
            for gs, gn in ((0, NL), (NL, NL), (2 * NL, CH - 2 * NL)):
                idv = idxbuf[pl.ds(gs, NL)]
                first = idv[0]
                last = idv[gn - 1]
                pid = prev_id

                def fast(first=first, last=last, pid=pid, gs=gs, gn=gn,
                         xbuf=xbuf):
                    @pl.when(first != pid)
                    def _():
                        pltpu.sync_copy(t_hbm.at[first], trowv)

                    tr = [trowv[pl.ds(c * NL, NL)] for c in range(NV)]

                    def rowfn(r, z):
                        d = [xbuf[r, pl.ds(j * NL, NL)] * tr[j]
                             for j in range(4)]
                        for c in range(4, NV):
                            j = c % 4
                            d[j] = d[j] + xbuf[r, pl.ds(c * NL, NL)] * tr[c]
                        dacc = (d[0] + d[1]) + (d[2] + d[3])
                        cfbuf[pl.ds(r * NL, NL)] = dot_gate(dacc)
                        return z

                    lax.fori_loop(gs, gs + gn, rowfn, 0, unroll=2)
                    return last

                def slow(pid=pid, gs=gs, gn=gn, xbuf=xbuf, idxbuf=idxbuf):
                    def rowfn(r, p):
                        rid = idxbuf[pl.ds(r, NL)][0]

                        @pl.when(rid != p)
                        def _():
                            pltpu.sync_copy(t_hbm.at[rid], trowv)

                        d = [(xbuf[r, pl.ds(j * NL, NL)]
                              * trowv[pl.ds(j * NL, NL)]) for j in range(4)]
                        for c in range(4, NV):
                            j = c % 4
                            d[j] = d[j] + (xbuf[r, pl.ds(c * NL, NL)]
                                           * trowv[pl.ds(c * NL, NL)])
                        dacc = (d[0] + d[1]) + (d[2] + d[3])
                        cfbuf[pl.ds(r * NL, NL)] = dot_gate(dacc)
                        return rid

                    return lax.fori_loop(gs, gs + gn, rowfn, pid)

                prev_id = lax.cond(first == last, fast, slow)
            return prev_id

        def phase_b(xbuf, idxbuf, carry):
            def row(r, rcarry):
                prev_id, *accs = rcarry
                rid = idxbuf[pl.ds(r, NL)][0]
                change = rid != prev_id

                @pl.when(jnp.logical_and(change, prev_id >= 0))
                def _():
                    flush(prev_id, accs)

                cf = cfbuf[pl.ds(r * NL, NL)]
                new_accs = []
                for c in range(NV):
                    wx = xbuf[r, pl.ds(c * NL, NL)] * cf
                    new_accs.append(jnp.where(change, wx, accs[c] + wx))
                return (rid, *new_accs)

            return lax.fori_loop(0, CH, row, carry, unroll=2)

        def compute(xbuf, idxbuf, carry):
            ca, cb = carry
            return (phase_a(xbuf, idxbuf, ca), phase_b(xbuf, idxbuf, cb))

        def pair(g, carry):
            rb0 = base + (2 * g) * CH
            rb1 = rb0 + CH
            _start_chunk(x_hbm, b_hbm, rb1, xbuf1, idx1, sem1)
            _wait_chunk(x_hbm, b_hbm, rb0, xbuf0, idx0, sem0)
            carry = compute(xbuf0, idx0, carry)

            @pl.when(g + 1 < npairs)
            def _():
                _start_chunk(x_hbm, b_hbm, rb1 + CH, xbuf0, idx0, sem0)

            _wait_chunk(x_hbm, b_hbm, rb1, xbuf1, idx1, sem1)
            return compute(xbuf1, idx1, carry)

        _start_chunk(x_hbm, b_hbm, base, xbuf0, idx0, sem0)
        zv16 = jnp.zeros((NL,), jnp.float32)
        init_b = (jnp.int32(-1),) + tuple(zv16 for _ in range(NV))
        carry_a, carry_b = lax.fori_loop(
            0, npairs, pair, (jnp.int32(-1), init_b))
        prev_id, *accs = carry_b

        @pl.when(prev_id >= 0)
        def _():
            flush(prev_id, accs)

    return k(x, batch, t)


def _tc_combine_call(parts, pcnt):
    def body(parts_ref, pcnt_ref, out_ref, acc):
        t = pl.program_id(0)

        @pl.when(t == 0)
        def _():
            acc[...] = jnp.zeros_like(acc)

        valid = pcnt_ref[0, :, 0] > 0.0
        acc[...] += jnp.where(valid[:, None], parts_ref[0], 0.0)

        @pl.when(t == NW - 1)
        def _():
            out_ref[...] = acc[...]

    return pl.pallas_call(
        body,
        grid=(NW,),
        in_specs=[
            pl.BlockSpec((1, S, D), lambda t: (t, 0, 0)),
            pl.BlockSpec((1, S, NL), lambda t: (t, 0, 0)),
        ],
        out_specs=pl.BlockSpec((S, D), lambda t: (0, 0)),
        out_shape=jax.ShapeDtypeStruct((S, D), jnp.float32),
        scratch_shapes=[pltpu.VMEM((S, D), jnp.float32)],
    )(parts, pcnt)


def kernel(x, batch, size, W):
    del size
    batch = batch.astype(jnp.int32)
    zero16 = jnp.zeros((S * NL,), jnp.float32)
    psum, pcnt = _sc_pass1_call(x, batch, zero16)
    pcnt = pcnt.reshape(NW, S, NL)
    t = _tc_dense_call(psum, pcnt, W)
    parts = _sc_pass2_call(x, batch, t)
    return _tc_combine_call(parts, pcnt)
